# Initial kernel scaffold; baseline (speedup 1.0000x reference)
import jax
import jax.numpy as jnp
from jax.experimental import pallas as pl


def kernel(x, edge_index, edge_weights, W1, b1, W2, b2, lin_w, lin_b):
    raise NotImplementedError("write your pallas kernel here")


# speedup vs baseline: 6.1216x; 6.1216x over previous
import functools

import jax
import jax.numpy as jnp
from jax import lax
from jax.experimental import pallas as pl
from jax.experimental.pallas import tpu as pltpu
from jax.experimental.pallas import tpu_sc as plsc

N = 10000
E = 320000
F_IN = 128
H = 30
C = 4
K = 5

HP = 32
NC = 2
NS = 16
NW = NC * NS
NP = 10240
RPS = NP // NS
VPS = RPS
EPW = E // NW
EPS = E // NS
EB = 80
NCH_W = EPW // EB
NCH_S = EPS // EB
BLK = 512
F32 = jnp.float32
I32 = jnp.int32


def _mesh():
    return plsc.VectorSubcoreMesh(core_axis_name="c", subcore_axis_name="s")


def _rsqrt16(d):
    bits = plsc.bitcast(d, I32)
    magic = jnp.full((16,), 0x5F3759DF, dtype=I32)
    y = plsc.bitcast(magic - lax.shift_right_logical(bits, 1), F32)
    half = 0.5 * d
    for _ in range(3):
        y = y * (1.5 - half * y * y)
    return jnp.where(d > 0.0, y, 0.0)


def _prep_body(row_h, col_h, ew_h, norm_h,
               deg_v, dis_v, acc_v, tmp_v, ir_v, ic_v, w_v, nrm_v,
               deg_sh, dis_sh):
    s = lax.axis_index("s")
    c = lax.axis_index("c")
    zero16 = jnp.zeros((16,), F32)

    def z_body(i, _):
        deg_v[pl.ds(i * 16, 16)] = zero16
        return 0
    lax.fori_loop(0, NP // 16, z_body, 0)

    def deg_chunk(ch, _):
        eb = pl.multiple_of((s * NCH_S + ch) * EB, 8)
        pltpu.sync_copy(row_h.at[pl.ds(eb, EB)], ir_v)
        pltpu.sync_copy(col_h.at[pl.ds(eb, EB)], ic_v)
        pltpu.sync_copy(ew_h.at[pl.ds(eb, EB)], w_v)
        for g in range(EB // 16):
            r = ir_v[pl.ds(g * 16, 16)]
            cc = ic_v[pl.ds(g * 16, 16)]
            w = w_v[pl.ds(g * 16, 16)]
            wm = jnp.where(r != cc, w, 0.0)
            plsc.addupdate_scatter(deg_v, [r], wm)
        return 0
    lax.fori_loop(0, NCH_S, deg_chunk, 0)

    pltpu.sync_copy(deg_v, deg_sh.at[s])
    plsc.subcore_barrier()

    rb = pl.multiple_of(s * RPS, 8)
    pltpu.sync_copy(deg_sh.at[0, pl.ds(rb, RPS)], acc_v)
    for t in range(1, NS):
        pltpu.sync_copy(deg_sh.at[t, pl.ds(rb, RPS)], tmp_v)

        def add_body(i, _):
            o = pl.ds(i * 16, 16)
            acc_v[o] = acc_v[o] + tmp_v[o]
            return 0
        lax.fori_loop(0, RPS // 16, add_body, 0)

    def rs_body(i, _):
        o = pl.ds(i * 16, 16)
        acc_v[o] = _rsqrt16(acc_v[o])
        return 0
    lax.fori_loop(0, RPS // 16, rs_body, 0)
    pltpu.sync_copy(acc_v, dis_sh.at[pl.ds(rb, RPS)])
    plsc.subcore_barrier()

    pltpu.sync_copy(dis_sh, dis_v)

    wid = s * NC + c
    def nrm_chunk(ch, _):
        eb = pl.multiple_of((wid * NCH_W + ch) * EB, 8)
        pltpu.sync_copy(row_h.at[pl.ds(eb, EB)], ir_v)
        pltpu.sync_copy(col_h.at[pl.ds(eb, EB)], ic_v)
        pltpu.sync_copy(ew_h.at[pl.ds(eb, EB)], w_v)
        for g in range(EB // 16):
            o = pl.ds(g * 16, 16)
            r = ir_v[o]
            cc = ic_v[o]
            w = w_v[o]
            dr = plsc.load_gather(dis_v, [r])
            dc = plsc.load_gather(dis_v, [cc])
            nv = jnp.where(r != cc, -(dr * w) * dc, 0.0)
            nrm_v[o] = nv
        pltpu.sync_copy(nrm_v, norm_h.at[pl.ds(eb, EB)])
        return 0
    lax.fori_loop(0, NCH_W, nrm_chunk, 0)


@functools.partial(jax.jit, static_argnums=())
def _sc_prep(row, col, ew):
    k = pl.kernel(
        _prep_body,
        out_type=jax.ShapeDtypeStruct((E,), F32),
        mesh=_mesh(),
        compiler_params=pltpu.CompilerParams(needs_layout_passes=False, use_tc_tiling_on_sc=False),
        scratch_types=[
            pltpu.VMEM((NP,), F32),
            pltpu.VMEM((NP,), F32),
            pltpu.VMEM((RPS,), F32),
            pltpu.VMEM((RPS,), F32),
            pltpu.VMEM((EB,), I32),
            pltpu.VMEM((EB,), I32),
            pltpu.VMEM((EB,), F32),
            pltpu.VMEM((EB,), F32),
            pltpu.VMEM_SHARED((NS, NP), F32),
            pltpu.VMEM_SHARED((NP,), F32),
        ],
    )
    return k(row, col, ew)


def _make_prop_body(first, has_prev):
    def body(*refs):
        it = iter(refs)
        z_h = next(it)
        accp_h = None if first else next(it)
        bprev_h = next(it) if has_prev else None
        row_h = next(it)
        col_h = next(it)
        nrm_h = next(it)
        acc_o = next(it)
        b_o = None if first else next(it)
        bufZ = next(it)
        bufA = None if first else next(it)
        bufB = None if first else next(it)
        bufP = next(it) if has_prev else None
        zb = next(it)
        ir_v = next(it)
        ic_v = next(it)
        nv_v = next(it)
        rows_v = next(it)
        b_sh = next(it)
        acc_sh = next(it)
        sem = next(it)

        s = lax.axis_index("s")
        c = lax.axis_index("c")
        zero16 = jnp.zeros((16,), F32)
        rb = pl.multiple_of(s * RPS, 8)

        pltpu.sync_copy(z_h.at[pl.ds(rb, RPS)], bufZ)
        if not first:
            pltpu.sync_copy(accp_h.at[0, pl.ds(rb, RPS)], bufA)
            pltpu.sync_copy(accp_h.at[1, pl.ds(rb, RPS)], bufB)
        if has_prev:
            pltpu.sync_copy(bprev_h.at[pl.ds(rb, RPS)], bufP)

        if not first:
            def comb(i, _):
                for h in range(2):
                    v = bufZ[i, h] + 2.0 * (bufA[i, h] + bufB[i, h])
                    if has_prev:
                        v = v - bufP[i, h]
                    bufZ[i, h] = v
                return 0
            lax.fori_loop(0, RPS, comb, 0)

        pltpu.sync_copy(bufZ, b_sh.at[pl.ds(rb, RPS)])
        if not first:
            @pl.when(c == 0)
            def _():
                pltpu.sync_copy(bufZ, b_o.at[pl.ds(rb, RPS)])

        def zloop(i, _):
            for h in range(2):
                zb[i, h] = zero16
            return 0
        lax.fori_loop(0, EB, zloop, 0)
        for q in range(RPS // EB):
            pltpu.sync_copy(zb, acc_sh.at[pl.ds(rb + q * EB, EB)])
        plsc.subcore_barrier()

        wid = s * NC + c

        def echunk(ch, _):
            eb = pl.multiple_of((wid * NCH_W + ch) * EB, 8)
            pltpu.sync_copy(row_h.at[pl.ds(eb, EB)], ir_v)
            pltpu.async_copy(b_sh.at[ir_v], rows_v, sem).wait()
            pltpu.sync_copy(nrm_h.at[pl.ds(eb, EB)], nv_v)
            for g in range(EB // 16):
                nvec = nv_v[pl.ds(g * 16, 16)]
                for j in range(16):
                    i = g * 16 + j
                    sc = nvec[j]
                    rows_v[i, 0] = rows_v[i, 0] * sc
                    rows_v[i, 1] = rows_v[i, 1] * sc
            pltpu.sync_copy(col_h.at[pl.ds(eb, EB)], ic_v)
            pltpu.sync_copy(rows_v, acc_sh.at[ic_v], add=True)
            return 0
        lax.fori_loop(0, NCH_W, echunk, 0)
        plsc.subcore_barrier()

        pltpu.sync_copy(acc_sh.at[pl.ds(rb, RPS)], acc_o.at[c, pl.ds(rb, RPS)])

    return body


def _make_prop(first, has_prev):
    n32 = jax.ShapeDtypeStruct((NP, 2, 16), F32)
    out_type = [jax.ShapeDtypeStruct((NC, NP, 2, 16), F32)]
    if not first:
        out_type.append(n32)
    scratch = [pltpu.VMEM((RPS, 2, 16), F32)]
    if not first:
        scratch += [pltpu.VMEM((RPS, 2, 16), F32),
                    pltpu.VMEM((RPS, 2, 16), F32)]
    if has_prev:
        scratch += [pltpu.VMEM((RPS, 2, 16), F32)]
    scratch += [
        pltpu.VMEM((EB, 2, 16), F32),
        pltpu.VMEM((EB,), I32),
        pltpu.VMEM((EB,), I32),
        pltpu.VMEM((EB,), F32),
        pltpu.VMEM((EB, 2, 16), F32),
        pltpu.VMEM_SHARED((NP, 2, 16), F32),
        pltpu.VMEM_SHARED((NP, 2, 16), F32),
        pltpu.SemaphoreType.DMA,
    ]
    return pl.kernel(
        _make_prop_body(first, has_prev),
        out_type=out_type,
        mesh=_mesh(),
        compiler_params=pltpu.CompilerParams(needs_layout_passes=False, use_tc_tiling_on_sc=False),
        scratch_types=scratch,
    )


def _tc_z1_body(x_ref, w_ref, *out_refs):
    acc = jnp.dot(x_ref[...], w_ref[...], preferred_element_type=F32)
    for k in range(K):
        out_refs[k][...] = acc[:, k * HP:(k + 1) * HP]


def _tc_z1(xp, wcat):
    return pl.pallas_call(
        _tc_z1_body,
        grid=(NP // BLK,),
        in_specs=[
            pl.BlockSpec((BLK, F_IN), lambda i: (i, 0)),
            pl.BlockSpec((F_IN, K * HP), lambda i: (0, 0)),
        ],
        out_specs=[pl.BlockSpec((BLK, HP), lambda i: (i, 0))] * K,
        out_shape=[jax.ShapeDtypeStruct((NP, HP), F32)] * K,
    )(xp, wcat)


def _tc_z2_body(z0_ref, a0_ref, a1_ref, b2_ref, bias_ref, w_ref, *out_refs):
    h = z0_ref[...] + (a0_ref[...] + a1_ref[...]) - b2_ref[...] + bias_ref[...]
    h = jnp.maximum(h, 0.0)
    acc = jnp.dot(h, w_ref[...], preferred_element_type=F32)
    for k in range(K):
        out_refs[k][...] = acc[:, k * HP:(k + 1) * HP]


def _tc_z2(z0, a0, a1, b2, bias, wcat):
    return pl.pallas_call(
        _tc_z2_body,
        grid=(NP // BLK,),
        in_specs=[
            pl.BlockSpec((BLK, HP), lambda i: (i, 0)),
            pl.BlockSpec((BLK, HP), lambda i: (i, 0)),
            pl.BlockSpec((BLK, HP), lambda i: (i, 0)),
            pl.BlockSpec((BLK, HP), lambda i: (i, 0)),
            pl.BlockSpec((1, HP), lambda i: (0, 0)),
            pl.BlockSpec((HP, K * HP), lambda i: (0, 0)),
        ],
        out_specs=[pl.BlockSpec((BLK, HP), lambda i: (i, 0))] * K,
        out_shape=[jax.ShapeDtypeStruct((NP, HP), F32)] * K,
    )(z0, a0, a1, b2, bias, wcat)


def _tc_final_body(z0_ref, a0_ref, a1_ref, b2_ref, bias_ref, w_ref, lb_ref,
                   out_ref):
    h = z0_ref[...] + (a0_ref[...] + a1_ref[...]) - b2_ref[...] + bias_ref[...]
    h = jnp.maximum(h, 0.0)
    out_ref[...] = jnp.dot(h, w_ref[...],
                           preferred_element_type=F32) + lb_ref[...]


def _tc_final(z0, a0, a1, b2, bias, wp, lbp):
    return pl.pallas_call(
        _tc_final_body,
        grid=(NP // BLK,),
        in_specs=[
            pl.BlockSpec((BLK, HP), lambda i: (i, 0)),
            pl.BlockSpec((BLK, HP), lambda i: (i, 0)),
            pl.BlockSpec((BLK, HP), lambda i: (i, 0)),
            pl.BlockSpec((BLK, HP), lambda i: (i, 0)),
            pl.BlockSpec((1, HP), lambda i: (0, 0)),
            pl.BlockSpec((HP, F_IN), lambda i: (0, 0)),
            pl.BlockSpec((1, F_IN), lambda i: (0, 0)),
        ],
        out_specs=pl.BlockSpec((BLK, F_IN), lambda i: (i, 0)),
        out_shape=jax.ShapeDtypeStruct((NP, F_IN), F32),
    )(z0, a0, a1, b2, bias, wp, lbp)


def _v(a):
    return a.reshape(NP, 2, 16)


def _cheb_layer(z, row, col, norm, prop_first, prop_mid, prop_full):
    z0, z1, z2, z3, z4 = z
    a4, = prop_first(_v(z4), row, col, norm)
    a3, b3 = prop_mid(_v(z3), a4, row, col, norm)
    a2, b2 = prop_full(_v(z2), a3, _v(z4), row, col, norm)
    a1, _ = prop_full(_v(z1), a2, b3, row, col, norm)
    acc = a1.reshape(NC, NP, HP)
    return acc[0], acc[1], b2.reshape(NP, HP)


def kernel(x, edge_index, edge_weights, W1, b1, W2, b2, lin_w, lin_b):
    row = edge_index[0]
    col = edge_index[1]

    xp = jnp.pad(x, ((0, NP - N), (0, 0)))
    w1cat = jnp.concatenate(
        [jnp.pad(W1[k], ((0, 0), (0, HP - H))) for k in range(K)], axis=1)
    w2cat = jnp.concatenate(
        [jnp.pad(W2[k], ((0, HP - H), (0, HP - H))) for k in range(K)], axis=1)
    bias1 = jnp.pad(b1, (0, HP - H)).reshape(1, HP)
    bias2 = jnp.pad(b2, (0, HP - H)).reshape(1, HP)
    linwp = jnp.pad(lin_w, ((0, HP - H), (0, F_IN - C)))
    linbp = jnp.pad(lin_b, (0, F_IN - C)).reshape(1, F_IN)

    prop_first = _make_prop(True, False)
    prop_mid = _make_prop(False, False)
    prop_full = _make_prop(False, True)

    norm = _sc_prep(row, col, edge_weights)

    z1s = _tc_z1(xp, w1cat)
    a0, a1, b2_1 = _cheb_layer(z1s, row, col, norm,
                               prop_first, prop_mid, prop_full)
    z2s = _tc_z2(z1s[0], a0, a1, b2_1, bias1, w2cat)
    c0, c1, b2_2 = _cheb_layer(z2s, row, col, norm,
                               prop_first, prop_mid, prop_full)
    out = _tc_final(z2s[0], c0, c1, b2_2, bias2, linwp, linbp)
    return out[:N, :C]


# speedup vs baseline: 19.2391x; 3.1428x over previous
import functools

import jax
import jax.numpy as jnp
from jax import lax
from jax.experimental import pallas as pl
from jax.experimental.pallas import tpu as pltpu
from jax.experimental.pallas import tpu_sc as plsc

N = 10000
E = 320000
F_IN = 128
H = 30
C = 4
K = 5

HP = 32
NC = 2
NS = 16
NW = NC * NS
NP = 10240
RPS = NP // NS
EPW = E // NW
EPS = E // NS
EB = 80
ECH = E // EB
NCH_W = EPW // EB
NCH_S = EPS // EB
BLK = 512
F32 = jnp.float32
I32 = jnp.int32

_CP = functools.partial(pltpu.CompilerParams,
                        needs_layout_passes=False, use_tc_tiling_on_sc=False)


def _mesh():
    return plsc.VectorSubcoreMesh(core_axis_name="c", subcore_axis_name="s")


def _rsqrt16(d):
    bits = plsc.bitcast(d, I32)
    magic = jnp.full((16,), 0x5F3759DF, dtype=I32)
    y = plsc.bitcast(magic - lax.shift_right_logical(bits, 1), F32)
    half = 0.5 * d
    for _ in range(3):
        y = y * (1.5 - half * y * y)
    return jnp.where(d > 0.0, y, 0.0)


def _prep_body(row_h, col_h, ew_h, norm_h,
               deg_v, dis_v, acc_v, tmp_v, rbuf, cbuf, wbuf, nrmbuf,
               deg_sh, dis_sh):
    s = lax.axis_index("s")
    c = lax.axis_index("c")
    zero16 = jnp.zeros((16,), F32)

    def z_body(i, _):
        deg_v[pl.ds(i * 16, 16)] = zero16
        return 0
    lax.fori_loop(0, NP // 16, z_body, 0)

    pltpu.sync_copy(row_h.at[pl.ds(s * NCH_S, NCH_S)], rbuf)
    pltpu.sync_copy(col_h.at[pl.ds(s * NCH_S, NCH_S)], cbuf)
    pltpu.sync_copy(ew_h.at[pl.ds(s * NCH_S, NCH_S)], wbuf)

    def deg_chunk(ch, _):
        for g in range(EB // 16):
            o = pl.ds(g * 16, 16)
            r = rbuf[ch, o]
            cc = cbuf[ch, o]
            w = wbuf[ch, o]
            wm = jnp.where(r != cc, w, 0.0)
            plsc.addupdate_scatter(deg_v, [r], wm)
        return 0
    lax.fori_loop(0, NCH_S, deg_chunk, 0)

    pltpu.sync_copy(deg_v, deg_sh.at[s])
    plsc.subcore_barrier()

    rb = pl.multiple_of(s * RPS, 8)
    pltpu.sync_copy(deg_sh.at[0, pl.ds(rb, RPS)], acc_v)
    for t in range(1, NS):
        pltpu.sync_copy(deg_sh.at[t, pl.ds(rb, RPS)], tmp_v)

        def add_body(i, _):
            o = pl.ds(i * 16, 16)
            acc_v[o] = acc_v[o] + tmp_v[o]
            return 0
        lax.fori_loop(0, RPS // 16, add_body, 0)

    def rs_body(i, _):
        o = pl.ds(i * 16, 16)
        acc_v[o] = _rsqrt16(acc_v[o])
        return 0
    lax.fori_loop(0, RPS // 16, rs_body, 0)
    pltpu.sync_copy(acc_v, dis_sh.at[pl.ds(rb, RPS)])
    plsc.subcore_barrier()

    pltpu.sync_copy(dis_sh, dis_v)

    wid = s * NC + c
    wb = wid * NCH_W
    pltpu.sync_copy(row_h.at[pl.ds(wb, NCH_W)], rbuf.at[pl.ds(0, NCH_W)])
    pltpu.sync_copy(col_h.at[pl.ds(wb, NCH_W)], cbuf.at[pl.ds(0, NCH_W)])
    pltpu.sync_copy(ew_h.at[pl.ds(wb, NCH_W)], wbuf.at[pl.ds(0, NCH_W)])

    def nrm_chunk(ch, _):
        for g in range(EB // 16):
            o = pl.ds(g * 16, 16)
            r = rbuf[ch, o]
            cc = cbuf[ch, o]
            w = wbuf[ch, o]
            dr = plsc.load_gather(dis_v, [r])
            dc = plsc.load_gather(dis_v, [cc])
            nrmbuf[ch, o] = jnp.where(r != cc, -(dr * w) * dc, 0.0)
        return 0
    lax.fori_loop(0, NCH_W, nrm_chunk, 0)
    pltpu.sync_copy(nrmbuf, norm_h.at[pl.ds(wb, NCH_W)])


def _sc_prep(row, col, ew):
    k = pl.kernel(
        _prep_body,
        out_type=jax.ShapeDtypeStruct((ECH, EB), F32),
        mesh=_mesh(),
        compiler_params=_CP(),
        scratch_types=[
            pltpu.VMEM((NP,), F32),
            pltpu.VMEM((NP,), F32),
            pltpu.VMEM((RPS,), F32),
            pltpu.VMEM((RPS,), F32),
            pltpu.VMEM((NCH_S, EB), I32),
            pltpu.VMEM((NCH_S, EB), I32),
            pltpu.VMEM((NCH_S, EB), F32),
            pltpu.VMEM((NCH_W, EB), F32),
            pltpu.VMEM_SHARED((NS, NP), F32),
            pltpu.VMEM_SHARED((NP,), F32),
        ],
    )
    return k(row, col, ew)


def _make_prop_body(first, has_prev):
    def body(*refs):
        it = iter(refs)
        z_h = next(it)
        accp_h = None if first else next(it)
        bprev_h = next(it) if has_prev else None
        row_h = next(it)
        col_h = next(it)
        nrm_h = next(it)
        acc_o = next(it)
        b_o = None if first else next(it)
        bufZ = next(it)
        bufA = None if first else next(it)
        zb = next(it)
        rbuf = next(it)
        cbuf = next(it)
        nbuf = next(it)
        rows_a = next(it)
        rows_b = next(it)
        b_sh = next(it)
        acc_sh = next(it)
        gsem = next(it)

        s = lax.axis_index("s")
        c = lax.axis_index("c")
        zero16 = jnp.zeros((16,), F32)
        rb = pl.multiple_of(s * RPS, 8)
        wid = s * NC + c
        wb = wid * NCH_W

        pltpu.sync_copy(row_h.at[pl.ds(wb, NCH_W)], rbuf)
        pltpu.sync_copy(col_h.at[pl.ds(wb, NCH_W)], cbuf)
        pltpu.sync_copy(nrm_h.at[pl.ds(wb, NCH_W)], nbuf)

        pltpu.sync_copy(z_h.at[pl.ds(rb, RPS)], bufZ)

        def _axpy(alpha):
            def body(i, _):
                for h in range(2):
                    bufZ[i, h] = bufZ[i, h] + alpha * bufA[i, h]
                return 0
            lax.fori_loop(0, RPS, body, 0)

        if not first:
            pltpu.sync_copy(accp_h.at[0, pl.ds(rb, RPS)], bufA)
            _axpy(2.0)
            pltpu.sync_copy(accp_h.at[1, pl.ds(rb, RPS)], bufA)
            _axpy(2.0)
        if has_prev:
            pltpu.sync_copy(bprev_h.at[pl.ds(rb, RPS)], bufA)
            _axpy(-1.0)

        pltpu.sync_copy(bufZ, b_sh.at[pl.ds(rb, RPS)])
        if not first:
            @pl.when(c == 0)
            def _():
                pltpu.sync_copy(bufZ, b_o.at[pl.ds(rb, RPS)])

        def zloop(i, _):
            for h in range(2):
                zb[i, h] = zero16
            return 0
        lax.fori_loop(0, EB, zloop, 0)
        for q in range(RPS // EB):
            pltpu.sync_copy(zb, acc_sh.at[pl.ds(rb + q * EB, EB)])
        plsc.subcore_barrier()

        def scale(rows, ch):
            for g in range(EB // 16):
                nvec = nbuf[ch, pl.ds(g * 16, 16)]
                for j in range(16):
                    i = g * 16 + j
                    sc = nvec[j]
                    rows[i, 0] = rows[i, 0] * sc
                    rows[i, 1] = rows[i, 1] * sc

        pltpu.async_copy(b_sh.at[rbuf.at[0]], rows_a, gsem)

        def pipe(p, _):
            for par in range(2):
                ch = 2 * p + par
                rows = rows_a if par == 0 else rows_b
                other = rows_b if par == 0 else rows_a
                pltpu.make_async_copy(b_sh.at[rbuf.at[ch]], rows, gsem).wait()
                pltpu.async_copy(b_sh.at[rbuf.at[ch + 1]], other, gsem)
                scale(rows, ch)
                pltpu.sync_copy(rows, acc_sh.at[cbuf.at[ch]], add=True)
            return 0
        lax.fori_loop(0, (NCH_W - 1) // 2, pipe, 0)
        tl = NCH_W - 1
        pltpu.make_async_copy(b_sh.at[rbuf.at[tl]], rows_a, gsem).wait()
        scale(rows_a, tl)
        pltpu.sync_copy(rows_a, acc_sh.at[cbuf.at[tl]], add=True)
        plsc.subcore_barrier()

        pltpu.sync_copy(acc_sh.at[pl.ds(rb, RPS)], acc_o.at[c, pl.ds(rb, RPS)])

    return body


def _make_prop(first, has_prev):
    n32 = jax.ShapeDtypeStruct((NP, 2, 16), F32)
    out_type = [jax.ShapeDtypeStruct((NC, NP, 2, 16), F32)]
    if not first:
        out_type.append(n32)
    scratch = [pltpu.VMEM((RPS, 2, 16), F32)]
    if not first:
        scratch += [pltpu.VMEM((RPS, 2, 16), F32)]
    scratch += [
        pltpu.VMEM((EB, 2, 16), F32),
        pltpu.VMEM((NCH_W, EB), I32),
        pltpu.VMEM((NCH_W, EB), I32),
        pltpu.VMEM((NCH_W, EB), F32),
        pltpu.VMEM((EB, 2, 16), F32),
        pltpu.VMEM((EB, 2, 16), F32),
        pltpu.VMEM_SHARED((NP, 2, 16), F32),
        pltpu.VMEM_SHARED((NP, 2, 16), F32),
        pltpu.SemaphoreType.DMA,
    ]
    return pl.kernel(
        _make_prop_body(first, has_prev),
        out_type=out_type,
        mesh=_mesh(),
        compiler_params=_CP(),
        scratch_types=scratch,
    )


def _tc_z1_body(x_ref, w_ref, *out_refs):
    acc = jnp.dot(x_ref[...], w_ref[...], preferred_element_type=F32)
    for k in range(K):
        out_refs[k][...] = acc[:, k * HP:(k + 1) * HP]


def _tc_z1(xp, wcat):
    return pl.pallas_call(
        _tc_z1_body,
        grid=(NP // BLK,),
        in_specs=[
            pl.BlockSpec((BLK, F_IN), lambda i: (i, 0)),
            pl.BlockSpec((F_IN, K * HP), lambda i: (0, 0)),
        ],
        out_specs=[pl.BlockSpec((BLK, HP), lambda i: (i, 0))] * K,
        out_shape=[jax.ShapeDtypeStruct((NP, HP), F32)] * K,
    )(xp, wcat)


def _tc_z2_body(z0_ref, a0_ref, a1_ref, b2_ref, bias_ref, w_ref, *out_refs):
    h = z0_ref[...] + (a0_ref[...] + a1_ref[...]) - b2_ref[...] + bias_ref[...]
    h = jnp.maximum(h, 0.0)
    acc = jnp.dot(h, w_ref[...], preferred_element_type=F32)
    for k in range(K):
        out_refs[k][...] = acc[:, k * HP:(k + 1) * HP]


def _tc_z2(z0, a0, a1, b2, bias, wcat):
    return pl.pallas_call(
        _tc_z2_body,
        grid=(NP // BLK,),
        in_specs=[
            pl.BlockSpec((BLK, HP), lambda i: (i, 0)),
            pl.BlockSpec((BLK, HP), lambda i: (i, 0)),
            pl.BlockSpec((BLK, HP), lambda i: (i, 0)),
            pl.BlockSpec((BLK, HP), lambda i: (i, 0)),
            pl.BlockSpec((1, HP), lambda i: (0, 0)),
            pl.BlockSpec((HP, K * HP), lambda i: (0, 0)),
        ],
        out_specs=[pl.BlockSpec((BLK, HP), lambda i: (i, 0))] * K,
        out_shape=[jax.ShapeDtypeStruct((NP, HP), F32)] * K,
    )(z0, a0, a1, b2, bias, wcat)


def _tc_final_body(z0_ref, a0_ref, a1_ref, b2_ref, bias_ref, w_ref, lb_ref,
                   out_ref):
    h = z0_ref[...] + (a0_ref[...] + a1_ref[...]) - b2_ref[...] + bias_ref[...]
    h = jnp.maximum(h, 0.0)
    out_ref[...] = jnp.dot(h, w_ref[...],
                           preferred_element_type=F32) + lb_ref[...]


def _tc_final(z0, a0, a1, b2, bias, wp, lbp):
    return pl.pallas_call(
        _tc_final_body,
        grid=(NP // BLK,),
        in_specs=[
            pl.BlockSpec((BLK, HP), lambda i: (i, 0)),
            pl.BlockSpec((BLK, HP), lambda i: (i, 0)),
            pl.BlockSpec((BLK, HP), lambda i: (i, 0)),
            pl.BlockSpec((BLK, HP), lambda i: (i, 0)),
            pl.BlockSpec((1, HP), lambda i: (0, 0)),
            pl.BlockSpec((HP, F_IN), lambda i: (0, 0)),
            pl.BlockSpec((1, F_IN), lambda i: (0, 0)),
        ],
        out_specs=pl.BlockSpec((BLK, F_IN), lambda i: (i, 0)),
        out_shape=jax.ShapeDtypeStruct((NP, F_IN), F32),
    )(z0, a0, a1, b2, bias, wp, lbp)


def _v(a):
    return a.reshape(NP, 2, 16)


def _cheb_layer(z, row, col, norm, prop_first, prop_mid, prop_full):
    z0, z1, z2, z3, z4 = z
    a4, = prop_first(_v(z4), row, col, norm)
    a3, b3 = prop_mid(_v(z3), a4, row, col, norm)
    a2, b2 = prop_full(_v(z2), a3, _v(z4), row, col, norm)
    a1, _ = prop_full(_v(z1), a2, b3, row, col, norm)
    acc = a1.reshape(NC, NP, HP)
    return acc[0], acc[1], b2.reshape(NP, HP)


def kernel(x, edge_index, edge_weights, W1, b1, W2, b2, lin_w, lin_b):
    row = edge_index[0].reshape(ECH, EB)
    col = edge_index[1].reshape(ECH, EB)
    ew = edge_weights.reshape(ECH, EB)

    xp = jnp.pad(x, ((0, NP - N), (0, 0)))
    w1cat = jnp.concatenate(
        [jnp.pad(W1[k], ((0, 0), (0, HP - H))) for k in range(K)], axis=1)
    w2cat = jnp.concatenate(
        [jnp.pad(W2[k], ((0, HP - H), (0, HP - H))) for k in range(K)], axis=1)
    bias1 = jnp.pad(b1, (0, HP - H)).reshape(1, HP)
    bias2 = jnp.pad(b2, (0, HP - H)).reshape(1, HP)
    linwp = jnp.pad(lin_w, ((0, HP - H), (0, F_IN - C)))
    linbp = jnp.pad(lin_b, (0, F_IN - C)).reshape(1, F_IN)

    prop_first = _make_prop(True, False)
    prop_mid = _make_prop(False, False)
    prop_full = _make_prop(False, True)

    norm = _sc_prep(row, col, ew)

    z1s = _tc_z1(xp, w1cat)
    a0, a1, b2_1 = _cheb_layer(z1s, row, col, norm,
                               prop_first, prop_mid, prop_full)
    z2s = _tc_z2(z1s[0], a0, a1, b2_1, bias1, w2cat)
    c0, c1, b2_2 = _cheb_layer(z2s, row, col, norm,
                               prop_first, prop_mid, prop_full)
    out = _tc_final(z2s[0], c0, c1, b2_2, bias2, linwp, linbp)
    return out[:N, :C]


# speedup vs baseline: 32.0475x; 1.6657x over previous
import functools

import jax
import jax.numpy as jnp
from jax import lax
from jax.experimental import pallas as pl
from jax.experimental.pallas import tpu as pltpu
from jax.experimental.pallas import tpu_sc as plsc

N = 10000
E = 320000
F_IN = 128
H = 30
C = 4
K = 5

HP = 32
NC = 2
NS = 16
NW = NC * NS
NP = 10240
RPS = NP // NS
EPW = E // NW
EPS = E // NS
EB = 80
ECH = E // EB
NCH_W = EPW // EB
NCH_S = EPS // EB
BLK = 1024
F32 = jnp.float32
I32 = jnp.int32

_CP = functools.partial(pltpu.CompilerParams,
                        needs_layout_passes=False, use_tc_tiling_on_sc=False)


def _mesh():
    return plsc.VectorSubcoreMesh(core_axis_name="c", subcore_axis_name="s")


def _rsqrt16(d):
    bits = plsc.bitcast(d, I32)
    magic = jnp.full((16,), 0x5F3759DF, dtype=I32)
    y = plsc.bitcast(magic - lax.shift_right_logical(bits, 1), F32)
    half = 0.5 * d
    for _ in range(3):
        y = y * (1.5 - half * y * y)
    return jnp.where(d > 0.0, y, 0.0)


def _prep_body(row_h, col_h, ew_h, norm_h,
               deg_v, dis_v, acc_v, tmp_v, tmp2_v, rbuf, cbuf, wbuf, nrmbuf,
               deg_sh, dis_sh, psem_a, psem_b):
    s = lax.axis_index("s")
    c = lax.axis_index("c")
    zero16 = jnp.zeros((16,), F32)

    def z_body(i, _):
        deg_v[pl.ds(i * 16, 16)] = zero16
        return 0
    lax.fori_loop(0, NP // 16, z_body, 0)

    pltpu.sync_copy(row_h.at[pl.ds(s * NCH_S, NCH_S)], rbuf)
    pltpu.sync_copy(col_h.at[pl.ds(s * NCH_S, NCH_S)], cbuf)
    pltpu.sync_copy(ew_h.at[pl.ds(s * NCH_S, NCH_S)], wbuf)

    def deg_chunk(ch, _):
        for g in range(EB // 16):
            o = pl.ds(g * 16, 16)
            r = rbuf[ch, o]
            cc = cbuf[ch, o]
            w = wbuf[ch, o]
            wm = jnp.where(r != cc, w, 0.0)
            plsc.addupdate_scatter(deg_v, [r], wm)
        return 0
    lax.fori_loop(0, NCH_S, deg_chunk, 0)

    pltpu.sync_copy(deg_v, deg_sh.at[s])
    plsc.subcore_barrier()

    rb = pl.multiple_of(s * RPS, 8)
    pltpu.sync_copy(deg_sh.at[0, pl.ds(rb, RPS)], acc_v)
    tbufs = (tmp_v, tmp2_v)
    sems = (psem_a, psem_b)

    def t_issue(t):
        pltpu.async_copy(deg_sh.at[t, pl.ds(rb, RPS)], tbufs[t % 2],
                         sems[t % 2])

    def t_wait(t):
        pltpu.make_async_copy(deg_sh.at[t, pl.ds(rb, RPS)], tbufs[t % 2],
                              sems[t % 2]).wait()

    t_issue(1)
    for t in range(1, NS):
        t_wait(t)
        if t + 1 < NS:
            t_issue(t + 1)
        tb = tbufs[t % 2]

        def add_body(i, _):
            o = pl.ds(i * 16, 16)
            acc_v[o] = acc_v[o] + tb[o]
            return 0
        lax.fori_loop(0, RPS // 16, add_body, 0)

    def rs_body(i, _):
        o = pl.ds(i * 16, 16)
        acc_v[o] = _rsqrt16(acc_v[o])
        return 0
    lax.fori_loop(0, RPS // 16, rs_body, 0)
    pltpu.sync_copy(acc_v, dis_sh.at[pl.ds(rb, RPS)])
    plsc.subcore_barrier()

    pltpu.sync_copy(dis_sh, dis_v)

    wid = s * NC + c
    wb = wid * NCH_W
    pltpu.sync_copy(row_h.at[pl.ds(wb, NCH_W)], rbuf.at[pl.ds(0, NCH_W)])
    pltpu.sync_copy(col_h.at[pl.ds(wb, NCH_W)], cbuf.at[pl.ds(0, NCH_W)])
    pltpu.sync_copy(ew_h.at[pl.ds(wb, NCH_W)], wbuf.at[pl.ds(0, NCH_W)])

    def nrm_chunk(ch, _):
        for g in range(EB // 16):
            o = pl.ds(g * 16, 16)
            r = rbuf[ch, o]
            cc = cbuf[ch, o]
            w = wbuf[ch, o]
            dr = plsc.load_gather(dis_v, [r])
            dc = plsc.load_gather(dis_v, [cc])
            nrmbuf[ch, o] = jnp.where(r != cc, -2.0 * (dr * w) * dc, 0.0)
        return 0
    lax.fori_loop(0, NCH_W, nrm_chunk, 0)
    pltpu.sync_copy(nrmbuf, norm_h.at[pl.ds(wb, NCH_W)])


def _sc_prep(row, col, ew):
    k = pl.kernel(
        _prep_body,
        out_type=jax.ShapeDtypeStruct((ECH, EB), F32),
        mesh=_mesh(),
        compiler_params=_CP(),
        scratch_types=[
            pltpu.VMEM((NP,), F32),
            pltpu.VMEM((NP,), F32),
            pltpu.VMEM((RPS,), F32),
            pltpu.VMEM((RPS,), F32),
            pltpu.VMEM((RPS,), F32),
            pltpu.VMEM((NCH_S, EB), I32),
            pltpu.VMEM((NCH_S, EB), I32),
            pltpu.VMEM((NCH_S, EB), F32),
            pltpu.VMEM((NCH_W, EB), F32),
            pltpu.VMEM_SHARED((NS, NP), F32),
            pltpu.VMEM_SHARED((NP,), F32),
            pltpu.SemaphoreType.DMA,
            pltpu.SemaphoreType.DMA,
        ],
    )
    return k(row, col, ew)


def _make_prop_body(first, has_prev):
    def body(*refs):
        it = iter(refs)
        z_h = next(it)
        accp_h = None if first else next(it)
        bprev_h = next(it) if has_prev else None
        row_h = next(it)
        col_h = next(it)
        nrm_h = next(it)
        acc_o = next(it)
        b_o = None if first else next(it)
        bufZ = next(it)
        bufA = None if first else next(it)
        idbuf = None if first else next(it)
        zb = next(it)
        rbuf = next(it)
        cbuf = next(it)
        nbuf = next(it)
        rows_a = next(it)
        rows_b = next(it)
        rows_c = next(it)
        b_sh = next(it)
        acc_sh = next(it)
        gsem_a = next(it)
        gsem_b = next(it)
        gsem_c = next(it)
        ssem_a = next(it)
        ssem_b = next(it)
        ssem_c = next(it)

        s = lax.axis_index("s")
        c = lax.axis_index("c")
        zero16 = jnp.zeros((16,), F32)
        rb = pl.multiple_of(s * RPS, 8)
        wid = s * NC + c
        wb = wid * NCH_W

        pltpu.async_copy(row_h.at[pl.ds(wb, NCH_W)], rbuf, gsem_a)
        pltpu.async_copy(col_h.at[pl.ds(wb, NCH_W)], cbuf, gsem_b)
        pltpu.async_copy(nrm_h.at[pl.ds(wb, NCH_W)], nbuf, gsem_c)

        if has_prev:
            pltpu.async_copy(z_h.at[pl.ds(rb, RPS)], bufZ, ssem_a)
            pltpu.async_copy(bprev_h.at[pl.ds(rb, RPS)], bufA, ssem_b)
        else:
            pltpu.async_copy(z_h.at[pl.ds(rb, RPS)], b_sh.at[pl.ds(rb, RPS)],
                             ssem_a)

        def zloop(i, _):
            for h in range(2):
                zb[i, pl.ds(h * 16, 16)] = zero16
            return 0
        lax.fori_loop(0, EB, zloop, 0)
        if not first:
            iota16 = lax.iota(I32, 16)

            def idloop(i, _):
                idbuf[pl.ds(i * 16, 16)] = iota16 + (rb + i * 16)
                return 0
            lax.fori_loop(0, RPS // 16, idloop, 0)
        for q in range(RPS // EB):
            pltpu.sync_copy(zb, acc_sh.at[pl.ds(rb + q * EB, EB)])

        if has_prev:
            pltpu.make_async_copy(z_h.at[pl.ds(rb, RPS)], bufZ,
                                  ssem_a).wait()
            pltpu.make_async_copy(bprev_h.at[pl.ds(rb, RPS)], bufA,
                                  ssem_b).wait()

            def sub_body(i, _):
                for h in range(2):
                    o = pl.ds(h * 16, 16)
                    bufZ[i, o] = bufZ[i, o] - bufA[i, o]
                return 0
            lax.fori_loop(0, RPS, sub_body, 0)
            pltpu.sync_copy(bufZ, b_sh.at[pl.ds(rb, RPS)])
        else:
            pltpu.make_async_copy(z_h.at[pl.ds(rb, RPS)],
                                  b_sh.at[pl.ds(rb, RPS)], ssem_a).wait()
        if not first:
            pltpu.sync_copy(accp_h.at[0, pl.ds(rb, RPS)], bufA)
            pltpu.async_copy(bufA, b_sh.at[idbuf], ssem_a, add=True)
            pltpu.sync_copy(accp_h.at[1, pl.ds(rb, RPS)], bufZ)
            pltpu.make_async_copy(bufA, b_sh.at[idbuf], ssem_a).wait()
            pltpu.sync_copy(bufZ, b_sh.at[idbuf], add=True)

            @pl.when(c == 0)
            def _():
                pltpu.sync_copy(b_sh.at[pl.ds(rb, RPS)],
                                b_o.at[pl.ds(rb, RPS)])

        pltpu.make_async_copy(row_h.at[pl.ds(wb, NCH_W)], rbuf, gsem_a).wait()
        pltpu.make_async_copy(col_h.at[pl.ds(wb, NCH_W)], cbuf, gsem_b).wait()
        pltpu.make_async_copy(nrm_h.at[pl.ds(wb, NCH_W)], nbuf, gsem_c).wait()
        plsc.subcore_barrier()

        rows3 = (rows_a, rows_b, rows_c)
        gsems = (gsem_a, gsem_b, gsem_c)
        ssems = (ssem_a, ssem_b, ssem_c)

        def scale(rows, ch):
            for g in range(EB // 16):
                nvec = nbuf[ch, pl.ds(g * 16, 16)]
                for j in range(16):
                    i = g * 16 + j
                    sc = nvec[j]
                    rows[i, pl.ds(0, 16)] = rows[i, pl.ds(0, 16)] * sc
                    rows[i, pl.ds(16, 16)] = rows[i, pl.ds(16, 16)] * sc

        def g_issue(ch, x):
            pltpu.async_copy(b_sh.at[rbuf.at[ch]], rows3[x], gsems[x])

        def g_wait(ch, x):
            pltpu.make_async_copy(b_sh.at[rbuf.at[ch]], rows3[x],
                                  gsems[x]).wait()

        def s_issue(ch, x):
            pltpu.async_copy(rows3[x], acc_sh.at[cbuf.at[ch]], ssems[x],
                             add=True)

        def s_wait(ch, x):
            pltpu.make_async_copy(rows3[x], acc_sh.at[cbuf.at[ch]],
                                  ssems[x]).wait()

        def step(ch, x, y):
            g_wait(ch, x)
            scale(rows3[x], ch)
            s_issue(ch, x)
            if y is not None:
                s_wait(ch - 1, y)
                g_issue(ch + 2, y)

        g_issue(0, 0)
        g_issue(1, 1)
        g_wait(0, 0)
        scale(rows_a, 0)
        s_issue(0, 0)
        g_issue(2, 2)

        def pipe(t, _):
            for u in range(3):
                ch = 3 * t + 1 + u
                step(ch, (1 + u) % 3, u)
            return 0
        lax.fori_loop(0, (NCH_W - 5) // 3, pipe, 0)
        step(NCH_W - 4, (NCH_W - 4) % 3, (NCH_W - 2) % 3)
        step(NCH_W - 3, (NCH_W - 3) % 3, (NCH_W - 1) % 3)
        step(NCH_W - 2, (NCH_W - 2) % 3, None)
        step(NCH_W - 1, (NCH_W - 1) % 3, None)
        s_wait(NCH_W - 3, (NCH_W - 3) % 3)
        s_wait(NCH_W - 2, (NCH_W - 2) % 3)
        s_wait(NCH_W - 1, (NCH_W - 1) % 3)
        plsc.subcore_barrier()

        pltpu.sync_copy(acc_sh.at[pl.ds(rb, RPS)], acc_o.at[c, pl.ds(rb, RPS)])

    return body


def _make_prop(first, has_prev):
    n32 = jax.ShapeDtypeStruct((NP, HP), F32)
    out_type = [jax.ShapeDtypeStruct((NC, NP, HP), F32)]
    if not first:
        out_type.append(n32)
    scratch = [pltpu.VMEM((RPS, HP), F32)]
    if not first:
        scratch += [pltpu.VMEM((RPS, HP), F32),
                    pltpu.VMEM((RPS,), I32)]
    scratch += [
        pltpu.VMEM((EB, HP), F32),
        pltpu.VMEM((NCH_W, EB), I32),
        pltpu.VMEM((NCH_W, EB), I32),
        pltpu.VMEM((NCH_W, EB), F32),
        pltpu.VMEM((EB, HP), F32),
        pltpu.VMEM((EB, HP), F32),
        pltpu.VMEM((EB, HP), F32),
        pltpu.VMEM_SHARED((NP, HP), F32),
        pltpu.VMEM_SHARED((NP, HP), F32),
        pltpu.SemaphoreType.DMA,
        pltpu.SemaphoreType.DMA,
        pltpu.SemaphoreType.DMA,
        pltpu.SemaphoreType.DMA,
        pltpu.SemaphoreType.DMA,
        pltpu.SemaphoreType.DMA,
    ]
    return pl.kernel(
        _make_prop_body(first, has_prev),
        out_type=out_type,
        mesh=_mesh(),
        compiler_params=_CP(),
        scratch_types=scratch,
    )


def _tc_z1_body(x_ref, w_ref, *out_refs):
    acc = jnp.dot(x_ref[...], w_ref[...], preferred_element_type=F32)
    for k in range(K):
        out_refs[k][...] = acc[:, k * HP:(k + 1) * HP]


def _tc_z1(xp, wcat):
    return pl.pallas_call(
        _tc_z1_body,
        grid=(NP // BLK,),
        in_specs=[
            pl.BlockSpec((BLK, F_IN), lambda i: (i, 0)),
            pl.BlockSpec((F_IN, K * HP), lambda i: (0, 0)),
        ],
        out_specs=[pl.BlockSpec((BLK, HP), lambda i: (i, 0))] * K,
        out_shape=[jax.ShapeDtypeStruct((NP, HP), F32)] * K,
    )(xp, wcat)


def _tc_z2_body(z0_ref, acc_ref, b2_ref, bias_ref, w_ref, *out_refs):
    h = (z0_ref[...] + 0.5 * (acc_ref[0] + acc_ref[1]) - b2_ref[...]
         + bias_ref[...])
    h = jnp.maximum(h, 0.0)
    acc = jnp.dot(h, w_ref[...], preferred_element_type=F32)
    for k in range(K):
        out_refs[k][...] = acc[:, k * HP:(k + 1) * HP]


def _tc_z2(z0, accp, b2, bias, wcat):
    return pl.pallas_call(
        _tc_z2_body,
        grid=(NP // BLK,),
        in_specs=[
            pl.BlockSpec((BLK, HP), lambda i: (i, 0)),
            pl.BlockSpec((NC, BLK, HP), lambda i: (0, i, 0)),
            pl.BlockSpec((BLK, HP), lambda i: (i, 0)),
            pl.BlockSpec((1, HP), lambda i: (0, 0)),
            pl.BlockSpec((HP, K * HP), lambda i: (0, 0)),
        ],
        out_specs=[pl.BlockSpec((BLK, HP), lambda i: (i, 0))] * K,
        out_shape=[jax.ShapeDtypeStruct((NP, HP), F32)] * K,
    )(z0, accp, b2, bias, wcat)


def _tc_final_body(z0_ref, acc_ref, b2_ref, bias_ref, w_ref, lb_ref,
                   out_ref):
    h = (z0_ref[...] + 0.5 * (acc_ref[0] + acc_ref[1]) - b2_ref[...]
         + bias_ref[...])
    h = jnp.maximum(h, 0.0)
    out_ref[...] = jnp.dot(h, w_ref[...],
                           preferred_element_type=F32) + lb_ref[...]


def _tc_final(z0, accp, b2, bias, wp, lbp):
    return pl.pallas_call(
        _tc_final_body,
        grid=(NP // BLK,),
        in_specs=[
            pl.BlockSpec((BLK, HP), lambda i: (i, 0)),
            pl.BlockSpec((NC, BLK, HP), lambda i: (0, i, 0)),
            pl.BlockSpec((BLK, HP), lambda i: (i, 0)),
            pl.BlockSpec((1, HP), lambda i: (0, 0)),
            pl.BlockSpec((HP, F_IN), lambda i: (0, 0)),
            pl.BlockSpec((1, F_IN), lambda i: (0, 0)),
        ],
        out_specs=pl.BlockSpec((BLK, F_IN), lambda i: (i, 0)),
        out_shape=jax.ShapeDtypeStruct((NP, F_IN), F32),
    )(z0, accp, b2, bias, wp, lbp)


def _cheb_layer(z, row, col, norm, prop_first, prop_mid, prop_full):
    z0, z1, z2, z3, z4 = z
    a4, = prop_first(z4, row, col, norm)
    a3, b3 = prop_mid(z3, a4, row, col, norm)
    a2, b2 = prop_full(z2, a3, z4, row, col, norm)
    a1, _ = prop_full(z1, a2, b3, row, col, norm)
    return a1, b2


def kernel(x, edge_index, edge_weights, W1, b1, W2, b2, lin_w, lin_b):
    row = edge_index[0].reshape(ECH, EB)
    col = edge_index[1].reshape(ECH, EB)
    ew = edge_weights.reshape(ECH, EB)

    xp = jnp.pad(x, ((0, NP - N), (0, 0)))
    w1cat = jnp.concatenate(
        [jnp.pad(W1[k], ((0, 0), (0, HP - H))) for k in range(K)], axis=1)
    w2cat = jnp.concatenate(
        [jnp.pad(W2[k], ((0, HP - H), (0, HP - H))) for k in range(K)], axis=1)
    bias1 = jnp.pad(b1, (0, HP - H)).reshape(1, HP)
    bias2 = jnp.pad(b2, (0, HP - H)).reshape(1, HP)
    linwp = jnp.pad(lin_w, ((0, HP - H), (0, F_IN - C)))
    linbp = jnp.pad(lin_b, (0, F_IN - C)).reshape(1, F_IN)

    prop_first = _make_prop(True, False)
    prop_mid = _make_prop(False, False)
    prop_full = _make_prop(False, True)

    norm = _sc_prep(row, col, ew)

    z1s = _tc_z1(xp, w1cat)
    a_1, b2_1 = _cheb_layer(z1s, row, col, norm,
                            prop_first, prop_mid, prop_full)
    z2s = _tc_z2(z1s[0], a_1, b2_1, bias1, w2cat)
    a_2, b2_2 = _cheb_layer(z2s, row, col, norm,
                            prop_first, prop_mid, prop_full)
    out = _tc_final(z2s[0], a_2, b2_2, bias2, linwp, linbp)
    return out[:N, :C]
